# baseline (speedup 1.0000x reference)
import jax
import jax.numpy as jnp
from jax import lax
from jax.experimental import pallas as pl
from jax.experimental.pallas import tpu as pltpu
from jax.experimental.pallas import tpu_sc as plsc

D = 768
F = 3072
E = 8
T = 2048
K = 2
NPAIR = T * K
B = 128
NB = 40
PADDED = NB * B
TRASH = 8
NW = 32
CHUNK = 128
NWR = NW


def _gate_body(x_ref, wg_ref, bg_ref, wp_ref, pos_ref, meta_ref,
               gid_ref, vld_ref):
    x = x_ref[...]
    scores = jnp.dot(x, wg_ref[...], preferred_element_type=jnp.float32)
    scores = scores + bg_ref[...]
    io = lax.broadcasted_iota(jnp.int32, (T, E), 1).astype(jnp.float32)
    m1 = jnp.max(scores, axis=1, keepdims=True)
    a1 = jnp.min(jnp.where(scores == m1, io, float(E)), axis=1, keepdims=True)
    masked = jnp.where(io == a1, -jnp.inf, scores)
    m2 = jnp.max(masked, axis=1, keepdims=True)
    a2 = jnp.min(jnp.where(masked == m2, io, float(E)), axis=1, keepdims=True)
    w1 = 1.0 / (1.0 + jnp.exp(m2 - m1))
    w2 = 1.0 - w1
    wp_ref[...] = jnp.concatenate([w1, w2], axis=1)

    oh1 = (io == a1).astype(jnp.float32)
    oh2 = (io == a2).astype(jnp.float32)
    tot1 = jnp.sum(oh1, axis=0, keepdims=True)
    tot = tot1 + jnp.sum(oh2, axis=0, keepdims=True)
    pu = jnp.floor((tot + float(B - 1)) * (1.0 / B)) * float(B)
    e0 = lax.broadcasted_iota(jnp.int32, (E, E), 0).astype(jnp.float32)
    e1 = lax.broadcasted_iota(jnp.int32, (E, E), 1).astype(jnp.float32)
    triE = (e0 < e1).astype(jnp.float32)
    base = jnp.dot(pu, triE, preferred_element_type=jnp.float32)

    t0 = lax.broadcasted_iota(jnp.int32, (T, T), 0)
    t1 = lax.broadcasted_iota(jnp.int32, (T, T), 1)
    triT = (t1 < t0).astype(jnp.bfloat16)
    oh_cat = jnp.concatenate([oh1, oh2], axis=1).astype(jnp.bfloat16)
    pref = jnp.dot(triT, oh_cat, preferred_element_type=jnp.float32)
    pos0 = jnp.sum(oh1 * (base + pref[:, :E]), axis=1, keepdims=True)
    pos1 = jnp.sum(oh2 * (base + tot1 + pref[:, E:]), axis=1, keepdims=True)
    pos_ref[...] = jnp.concatenate([pos0, pos1], axis=1).astype(jnp.int32)

    tp = jnp.sum(pu)
    z8 = jnp.zeros((1, E), jnp.float32)
    row0 = jnp.concatenate([base + tot, z8], axis=1)
    row1 = jnp.concatenate([pu - tot, z8], axis=1)
    row2 = jnp.full((1, 2 * E), tp, jnp.float32)
    row3 = jnp.zeros((1, 2 * E), jnp.float32)
    meta_ref[...] = jnp.concatenate([row0, row1, row2, row3],
                                    axis=0).astype(jnp.int32)

    bs = lax.broadcasted_iota(jnp.int32, (48, E), 0).astype(jnp.float32) * float(B)
    pend = base + pu
    gid = jnp.sum((bs >= pend).astype(jnp.float32), axis=1, keepdims=True)
    gid = jnp.minimum(gid, float(E - 1))
    valid = (bs < tp).astype(jnp.float32)
    gid_last = jnp.max(jnp.where(valid > 0.0, jnp.broadcast_to(gid, (48, E)),
                                 0.0))
    gidf = jnp.where(valid > 0.0, jnp.broadcast_to(gid, (48, E)), gid_last)
    gid_ref[...] = gidf.astype(jnp.int32)
    vld_ref[...] = valid.astype(jnp.int32)


def _gate(x, Wg, bg2):
    out_shapes = (
        jax.ShapeDtypeStruct((T, K), jnp.float32),
        jax.ShapeDtypeStruct((T, K), jnp.int32),
        jax.ShapeDtypeStruct((4, 2 * E), jnp.int32),
        jax.ShapeDtypeStruct((48, E), jnp.int32),
        jax.ShapeDtypeStruct((48, E), jnp.int32),
    )
    return pl.pallas_call(_gate_body, out_shape=out_shapes)(x, Wg, bg2)


def _route_body(wp_hbm, pos_hbm, meta_hbm,
                src_hbm, ws_hbm,
                wchunk, posbuf, metav, tokbuf,
                zbi, zbf, idxbuf, sem):
    wid = lax.axis_index("s") * 2 + lax.axis_index("c")
    base_j = wid * CHUNK
    pltpu.sync_copy(pos_hbm.at[pl.ds(base_j, CHUNK)], posbuf)
    pltpu.sync_copy(wp_hbm.at[pl.ds(base_j, CHUNK)], wchunk)
    pltpu.sync_copy(meta_hbm, metav)

    lane = lax.iota(jnp.int32, 16)
    zero16i = jnp.zeros((16,), jnp.int32)
    for c in range(CHUNK // 16):
        sl = pl.ds(16 * c, 16)
        j16 = base_j + 16 * c + lane
        tokbuf[sl] = jnp.bitwise_and(j16, T - 1)
        zbi[sl] = zero16i
        zbf[sl] = jnp.zeros((16,), jnp.float32)

    pltpu.async_copy(tokbuf, src_hbm.at[posbuf], sem).wait()
    pltpu.async_copy(wchunk, ws_hbm.at[posbuf], sem).wait()

    @pl.when(wid < E)
    def _():
        r0 = metav[0]
        r1 = metav[1]
        ps = jnp.sum(jnp.where(lane == wid, r0, 0))
        pc_ = jnp.sum(jnp.where(lane == wid, r1, 0))
        for c in range(CHUNK // 16):
            off = 16 * c + lane
            idxbuf[pl.ds(16 * c, 16)] = jnp.where(off < pc_, ps + off,
                                                  PADDED + wid)
        pltpu.async_copy(zbi, src_hbm.at[idxbuf], sem).wait()
        pltpu.async_copy(zbf, ws_hbm.at[idxbuf], sem).wait()

    @pl.when(jnp.logical_and(wid >= E, wid < 2 * E))
    def _():
        q = wid - E
        r2 = metav[2]
        tp = jnp.sum(jnp.where(lane == 0, r2, 0))
        for c in range(CHUNK // 16):
            slot = tp + q * CHUNK + 16 * c + lane
            idxbuf[pl.ds(16 * c, 16)] = jnp.where(slot < PADDED, slot,
                                                  PADDED + q)
        pltpu.async_copy(zbi, src_hbm.at[idxbuf], sem).wait()
        pltpu.async_copy(zbf, ws_hbm.at[idxbuf], sem).wait()


def _route(wp_flat, pos_flat, padmeta):
    mesh = plsc.VectorSubcoreMesh(core_axis_name="c", subcore_axis_name="s")
    fn = pl.kernel(
        _route_body,
        out_type=(
            jax.ShapeDtypeStruct((PADDED + TRASH,), jnp.int32),
            jax.ShapeDtypeStruct((PADDED + TRASH,), jnp.float32),
        ),
        mesh=mesh,
        compiler_params=pltpu.CompilerParams(needs_layout_passes=False),
        scratch_types=[
            pltpu.VMEM((CHUNK,), jnp.float32),
            pltpu.VMEM((CHUNK,), jnp.int32),
            pltpu.VMEM((4, 2 * E), jnp.int32),
            pltpu.VMEM((CHUNK,), jnp.int32),
            pltpu.VMEM((CHUNK,), jnp.int32),
            pltpu.VMEM((CHUNK,), jnp.float32),
            pltpu.VMEM((CHUNK,), jnp.int32),
            pltpu.SemaphoreType.DMA,
        ],
    )
    return fn(wp_flat, pos_flat, padmeta)


def _gather_body(src_hbm, x_hbm, xs_hbm, idxv, rows, sem):
    wid = lax.axis_index("s") * 2 + lax.axis_index("c")
    per = PADDED // NW
    for r in range(2):
        b0 = wid * per + r * (per // 2)
        pltpu.sync_copy(src_hbm.at[pl.ds(b0, per // 2)], idxv)
        pltpu.async_copy(x_hbm.at[idxv], rows, sem).wait()
        pltpu.sync_copy(rows, xs_hbm.at[pl.ds(b0, per // 2)])


def _gather(src, x):
    mesh = plsc.VectorSubcoreMesh(core_axis_name="c", subcore_axis_name="s")
    fn = pl.kernel(
        _gather_body,
        out_type=jax.ShapeDtypeStruct((PADDED, D), jnp.float32),
        mesh=mesh,
        scratch_types=[
            pltpu.VMEM((PADDED // NW // 2,), jnp.int32),
            pltpu.VMEM((PADDED // NW // 2, D), jnp.float32),
            pltpu.SemaphoreType.DMA,
        ],
    )
    return fn(src, x)


def _ffn_body(gid_ref, vld_ref, xs_ref, w1_ref, b1_ref, w2_ref, b2_ref,
              ws_ref, out_ref):
    b = pl.program_id(0)

    @pl.when(vld_ref[b] == 1)
    def _():
        xv = xs_ref[...]
        h = jnp.dot(xv, w1_ref[0], preferred_element_type=jnp.float32)
        h = jnp.maximum(h + b1_ref[0], 0.0)
        y = jnp.dot(h, w2_ref[0], preferred_element_type=jnp.float32)
        y = y + b2_ref[0]
        i0 = lax.broadcasted_iota(jnp.int32, (B, B), 0).astype(jnp.float32)
        i1 = lax.broadcasted_iota(jnp.int32, (B, B), 1).astype(jnp.float32)
        ident = (i0 == i1).astype(jnp.float32)
        wcol = lax.dot_general(ident, ws_ref[0], (((1,), (1,)), ((), ())),
                               preferred_element_type=jnp.float32)
        out_ref[...] = y * wcol

    @pl.when(vld_ref[b] == 0)
    def _():
        out_ref[...] = jnp.zeros_like(out_ref)


def _ffn(gid, valid, xs, W1, b1r, W2, b2r, wsr):
    grid_spec = pltpu.PrefetchScalarGridSpec(
        num_scalar_prefetch=2,
        grid=(NB,),
        in_specs=[
            pl.BlockSpec((B, D), lambda b, g, v: (b, 0)),
            pl.BlockSpec((1, D, F), lambda b, g, v: (g[b], 0, 0)),
            pl.BlockSpec((1, 1, F), lambda b, g, v: (g[b], 0, 0)),
            pl.BlockSpec((1, F, D), lambda b, g, v: (g[b], 0, 0)),
            pl.BlockSpec((1, 1, D), lambda b, g, v: (g[b], 0, 0)),
            pl.BlockSpec((1, 1, B), lambda b, g, v: (b, 0, 0)),
        ],
        out_specs=pl.BlockSpec((B, D), lambda b, g, v: (b, 0)),
    )
    return pl.pallas_call(
        _ffn_body,
        grid_spec=grid_spec,
        out_shape=jax.ShapeDtypeStruct((PADDED, D), jnp.float32),
    )(gid, valid, xs, W1, b1r, W2, b2r, wsr)


def _combine_body(pos_hbm, ys_hbm, out_hbm, idxE, idxO, bufE, bufO, sem):
    wid = lax.axis_index("s") * 2 + lax.axis_index("c")
    per = T // NW
    t0 = wid * per
    pltpu.sync_copy(pos_hbm.at[pl.ds(t0, per)], idxE)
    pltpu.sync_copy(pos_hbm.at[pl.ds(T + t0, per)], idxO)
    pltpu.async_copy(ys_hbm.at[idxE], bufE, sem).wait()
    pltpu.async_copy(ys_hbm.at[idxO], bufO, sem).wait()

    def body(i, carry):
        for d in range(D // 16):
            sl = pl.ds(16 * d, 16)
            bufE[i, sl] = bufE[i, sl] + bufO[i, sl]
        return carry

    lax.fori_loop(0, per, body, 0)
    pltpu.sync_copy(bufE, out_hbm.at[pl.ds(t0, per)])


def _combine(pos, ys):
    mesh = plsc.VectorSubcoreMesh(core_axis_name="c", subcore_axis_name="s")
    fn = pl.kernel(
        _combine_body,
        out_type=jax.ShapeDtypeStruct((T, D), jnp.float32),
        mesh=mesh,
        scratch_types=[
            pltpu.VMEM((T // NW,), jnp.int32),
            pltpu.VMEM((T // NW,), jnp.int32),
            pltpu.VMEM((T // NW, D), jnp.float32),
            pltpu.VMEM((T // NW, D), jnp.float32),
            pltpu.SemaphoreType.DMA,
        ],
    )
    return fn(pos, ys)


@jax.jit
def kernel(x, Wg, bg, W1, b1, W2, b2):
    wp, pos, padmeta, gidv, validv = _gate(x, Wg, bg.reshape(1, E))
    wp_flat = wp.T.reshape(NPAIR)
    pos_flat = pos.T.reshape(NPAIR)
    gid = gidv[:NB, 0]
    valid = validv[:NB, 0]
    src, ws = _route(wp_flat, pos_flat, padmeta)
    src = src[:PADDED]
    ws = ws[:PADDED]
    xs = _gather(src, x)
    ys = _ffn(gid, valid, xs, W1, b1.reshape(E, 1, F), W2, b2.reshape(E, 1, D),
              ws.reshape(NB, 1, B))
    return _combine(pos_flat, ys)


# speedup vs baseline: 2.2791x; 2.2791x over previous
import jax
import jax.numpy as jnp
from jax import lax
from jax.experimental import pallas as pl
from jax.experimental.pallas import tpu as pltpu
from jax.experimental.pallas import tpu_sc as plsc

D = 768
F = 3072
E = 8
T = 2048
K = 2
NPAIR = T * K
B = 128
NB = 40
PADDED = NB * B
NW = 32
CHUNK = 128


def _gate_body(x_ref, wg_ref, bg_ref, wp_ref, pos_ref, gid_ref):
    x = x_ref[...]
    scores = jnp.dot(x, wg_ref[...], preferred_element_type=jnp.float32)
    scores = scores + bg_ref[...]
    io = lax.broadcasted_iota(jnp.int32, (T, E), 1).astype(jnp.float32)
    m1 = jnp.max(scores, axis=1, keepdims=True)
    a1 = jnp.min(jnp.where(scores == m1, io, float(E)), axis=1, keepdims=True)
    masked = jnp.where(io == a1, -jnp.inf, scores)
    m2 = jnp.max(masked, axis=1, keepdims=True)
    a2 = jnp.min(jnp.where(masked == m2, io, float(E)), axis=1, keepdims=True)
    w1 = 1.0 / (1.0 + jnp.exp(m2 - m1))
    w2 = 1.0 - w1
    wp_ref[...] = jnp.concatenate([w1, w2], axis=1)

    oh1 = (io == a1).astype(jnp.float32)
    oh2 = (io == a2).astype(jnp.float32)
    tot1 = jnp.sum(oh1, axis=0, keepdims=True)
    tot = tot1 + jnp.sum(oh2, axis=0, keepdims=True)
    pu = jnp.floor((tot + float(B - 1)) * (1.0 / B)) * float(B)
    e0 = lax.broadcasted_iota(jnp.int32, (E, E), 0).astype(jnp.float32)
    e1 = lax.broadcasted_iota(jnp.int32, (E, E), 1).astype(jnp.float32)
    triE = (e0 < e1).astype(jnp.float32)
    base = jnp.dot(pu, triE, preferred_element_type=jnp.float32)

    t0 = lax.broadcasted_iota(jnp.int32, (T, T), 0)
    t1 = lax.broadcasted_iota(jnp.int32, (T, T), 1)
    triT = (t1 < t0).astype(jnp.bfloat16)
    oh_cat = jnp.concatenate([oh1, oh2], axis=1).astype(jnp.bfloat16)
    pref = jnp.dot(triT, oh_cat, preferred_element_type=jnp.float32)
    pos0 = jnp.sum(oh1 * (base + pref[:, :E]), axis=1, keepdims=True)
    pos1 = jnp.sum(oh2 * (base + tot1 + pref[:, E:]), axis=1, keepdims=True)
    pos_ref[...] = jnp.concatenate([pos0, pos1], axis=1).astype(jnp.int32)

    tp = jnp.sum(pu)
    bs = lax.broadcasted_iota(jnp.int32, (48, E), 0).astype(jnp.float32) * float(B)
    pend = base + pu
    gid = jnp.sum((bs >= pend).astype(jnp.float32), axis=1, keepdims=True)
    gid = jnp.minimum(gid, float(E - 1))
    valid = (bs < tp).astype(jnp.float32)
    gid_last = jnp.max(jnp.where(valid > 0.0, jnp.broadcast_to(gid, (48, E)),
                                 0.0))
    gidf = jnp.where(valid > 0.0, jnp.broadcast_to(gid, (48, E)), gid_last)
    gid_ref[...] = gidf.astype(jnp.int32)


def _gate(x, Wg, bg2):
    out_shapes = (
        jax.ShapeDtypeStruct((T, K), jnp.float32),
        jax.ShapeDtypeStruct((T, K), jnp.int32),
        jax.ShapeDtypeStruct((48, E), jnp.int32),
    )
    return pl.pallas_call(_gate_body, out_shape=out_shapes)(x, Wg, bg2)


def _dispatch_body(pos_hbm, x_hbm, xs_hbm, posbuf0, posbuf1, rows, sem):
    wid = lax.axis_index("s") * 2 + lax.axis_index("c")
    base_j = pl.multiple_of(wid * CHUNK, CHUNK)
    tok0 = pl.multiple_of(base_j & (T - 1), CHUNK)
    half = CHUNK // 2
    pltpu.sync_copy(pos_hbm.at[pl.ds(base_j, half)], posbuf0)
    pltpu.sync_copy(pos_hbm.at[pl.ds(base_j + half, half)], posbuf1)
    pltpu.sync_copy(x_hbm.at[pl.ds(tok0, half)], rows)
    pltpu.async_copy(rows, xs_hbm.at[posbuf0], sem).wait()
    pltpu.sync_copy(x_hbm.at[pl.ds(tok0 + half, half)], rows)
    pltpu.async_copy(rows, xs_hbm.at[posbuf1], sem).wait()


def _dispatch(pos_flat, x):
    mesh = plsc.VectorSubcoreMesh(core_axis_name="c", subcore_axis_name="s")
    fn = pl.kernel(
        _dispatch_body,
        out_type=jax.ShapeDtypeStruct((PADDED, D), jnp.float32),
        mesh=mesh,
        scratch_types=[
            pltpu.VMEM((CHUNK // 2,), jnp.int32),
            pltpu.VMEM((CHUNK // 2,), jnp.int32),
            pltpu.VMEM((CHUNK // 2, D), jnp.float32),
            pltpu.SemaphoreType.DMA,
        ],
    )
    return fn(pos_flat, x)


def _ffn_body(gid_ref, xs_ref, w1_ref, b1_ref, w2_ref, b2_ref, out_ref):
    xv = xs_ref[...]
    h = jnp.dot(xv, w1_ref[0], preferred_element_type=jnp.float32)
    h = jnp.maximum(h + b1_ref[0], 0.0)
    y = jnp.dot(h, w2_ref[0], preferred_element_type=jnp.float32)
    out_ref[...] = y + b2_ref[0]


def _ffn(gid, xs, W1, b1r, W2, b2r):
    grid_spec = pltpu.PrefetchScalarGridSpec(
        num_scalar_prefetch=1,
        grid=(NB,),
        in_specs=[
            pl.BlockSpec((B, D), lambda b, g: (b, 0)),
            pl.BlockSpec((1, D, F), lambda b, g: (g[b], 0, 0)),
            pl.BlockSpec((1, 1, F), lambda b, g: (g[b], 0, 0)),
            pl.BlockSpec((1, F, D), lambda b, g: (g[b], 0, 0)),
            pl.BlockSpec((1, 1, D), lambda b, g: (g[b], 0, 0)),
        ],
        out_specs=pl.BlockSpec((B, D), lambda b, g: (b, 0)),
    )
    return pl.pallas_call(
        _ffn_body,
        grid_spec=grid_spec,
        out_shape=jax.ShapeDtypeStruct((PADDED, D), jnp.float32),
    )(gid, xs, W1, b1r, W2, b2r)


def _combine_body(pos_hbm, wp_hbm, ys_hbm, out_hbm,
                  idxE, idxO, wE, wO, bufE, bufO, sem):
    wid = lax.axis_index("s") * 2 + lax.axis_index("c")
    per = T // NW
    t0 = wid * per
    pltpu.sync_copy(pos_hbm.at[pl.ds(t0, per)], idxE)
    pltpu.sync_copy(pos_hbm.at[pl.ds(T + t0, per)], idxO)
    pltpu.sync_copy(wp_hbm.at[pl.ds(t0, per)], wE)
    pltpu.sync_copy(wp_hbm.at[pl.ds(T + t0, per)], wO)
    pltpu.async_copy(ys_hbm.at[idxE], bufE, sem).wait()
    pltpu.async_copy(ys_hbm.at[idxO], bufO, sem).wait()

    lane = lax.iota(jnp.int32, 16)

    def body(i, carry):
        g16 = pl.multiple_of((i // 16) * 16, 16)
        m = lane == (i & 15)
        we = jnp.sum(jnp.where(m, wE[pl.ds(g16, 16)], 0.0))
        wo = jnp.sum(jnp.where(m, wO[pl.ds(g16, 16)], 0.0))
        for d in range(D // 16):
            sl = pl.ds(16 * d, 16)
            bufE[i, sl] = we * bufE[i, sl] + wo * bufO[i, sl]
        return carry

    lax.fori_loop(0, per, body, 0)
    pltpu.sync_copy(bufE, out_hbm.at[pl.ds(t0, per)])


def _combine(pos, wp_flat, ys):
    mesh = plsc.VectorSubcoreMesh(core_axis_name="c", subcore_axis_name="s")
    fn = pl.kernel(
        _combine_body,
        out_type=jax.ShapeDtypeStruct((T, D), jnp.float32),
        mesh=mesh,
        compiler_params=pltpu.CompilerParams(needs_layout_passes=False),
        scratch_types=[
            pltpu.VMEM((T // NW,), jnp.int32),
            pltpu.VMEM((T // NW,), jnp.int32),
            pltpu.VMEM((T // NW,), jnp.float32),
            pltpu.VMEM((T // NW,), jnp.float32),
            pltpu.VMEM((T // NW, D), jnp.float32),
            pltpu.VMEM((T // NW, D), jnp.float32),
            pltpu.SemaphoreType.DMA,
        ],
    )
    return fn(pos, wp_flat, ys)


@jax.jit
def kernel(x, Wg, bg, W1, b1, W2, b2):
    wp, pos, gidv = _gate(x, Wg, bg.reshape(1, E))
    wp_flat = wp.T.reshape(NPAIR)
    pos_flat = pos.T.reshape(NPAIR)
    gid = gidv[:NB, 0]
    xs = _dispatch(pos_flat, x)
    ys = _ffn(gid, xs, W1, b1.reshape(E, 1, F), W2, b2.reshape(E, 1, D))
    return _combine(pos_flat, wp_flat, ys)
